# baseline (speedup 1.0000x reference)
import functools

import jax
import jax.numpy as jnp
from jax import lax
from jax.experimental import pallas as pl
from jax.experimental.pallas import tpu as pltpu
from jax.experimental.pallas import tpu_sc as plsc


def _sc_gather(table, idx):
    NB, NS_ = idx.shape
    NT = NB * NS_
    D = table.shape[1]
    info = plsc.get_sparse_core_info()
    NC, NS = info.num_cores, info.num_subcores
    NW = NC * NS
    per_w = NT // NW
    CH = 64
    n_ch = per_w // CH

    mesh = plsc.VectorSubcoreMesh(core_axis_name="c", subcore_axis_name="s")

    @functools.partial(
        pl.kernel,
        mesh=mesh,
        out_type=jax.ShapeDtypeStruct((NT, D), jnp.float32),
        scratch_types=(
            [pltpu.VMEM((per_w,), jnp.int32)]
            + [pltpu.VMEM((CH, D), jnp.float32)] * n_ch
            + [pltpu.SemaphoreType.DMA] * (2 * n_ch)
        ),
    )
    def gk(idx_hbm, table_hbm, out_hbm, idx_v, *scr):
        rows = scr[:n_ch]
        sg = scr[n_ch:2 * n_ch]
        sw = scr[2 * n_ch:]
        wid = lax.axis_index("s") * NC + lax.axis_index("c")
        base = wid * per_w
        row = base // NS_
        col = base - row * NS_
        pltpu.sync_copy(idx_hbm.at[row, pl.ds(col, per_w)], idx_v)
        gathers = [
            pltpu.async_copy(table_hbm.at[idx_v.at[pl.ds(j * CH, CH)]],
                             rows[j], sg[j])
            for j in range(n_ch)
        ]
        writes = []
        for j in range(n_ch):
            gathers[j].wait()
            writes.append(pltpu.async_copy(
                rows[j], out_hbm.at[pl.ds(base + j * CH, CH)], sw[j]))
        for w in writes:
            w.wait()

    return gk(idx, table)


def _tc_body(w_ref, tt_ref, pw_ref, pb_ref, pos_ref, tb_ref, g_ref, bt_ref,
             o_ref):
    B, BSS, E = w_ref.shape
    H = pos_ref.shape[-1]
    x = jnp.dot(w_ref[...].reshape(B * BSS, E), pw_ref[...],
                preferred_element_type=jnp.float32).reshape(B, BSS, H)
    x = x + pb_ref[...] + pos_ref[...][None]
    tid = tt_ref[0].astype(jnp.float32)
    x = x + tb_ref[0:1, :] + tid * (tb_ref[1:2, :] - tb_ref[0:1, :])
    mean = jnp.mean(x, axis=2, keepdims=True)
    xc = x - mean
    var = jnp.mean(xc * xc, axis=2, keepdims=True)
    inv = lax.rsqrt(var + 1e-5)
    o_ref[...] = (xc * inv) * g_ref[...] + bt_ref[...]


def kernel(input_ids, token_type_ids, word_table, proj_W, proj_b,
           pos_table, type_table, ln_gamma, ln_beta):
    B, S = input_ids.shape
    V, E = word_table.shape
    H = proj_W.shape[1]
    BSS = 1024

    gathered = _sc_gather(word_table,
                          input_ids.astype(jnp.int32)).reshape(B, S, E)
    tt = (token_type_ids.astype(jnp.int8)
          .reshape(B, S // BSS, BSS).transpose(1, 0, 2)
          .reshape(S // BSS, B, BSS, 1))

    out = pl.pallas_call(
        _tc_body,
        grid=(S // BSS,),
        in_specs=[
            pl.BlockSpec((B, BSS, E), lambda s: (0, s, 0)),
            pl.BlockSpec((1, B, BSS, 1), lambda s: (s, 0, 0, 0)),
            pl.BlockSpec((E, H), lambda s: (0, 0)),
            pl.BlockSpec((H,), lambda s: (0,)),
            pl.BlockSpec((BSS, H), lambda s: (s, 0)),
            pl.BlockSpec((2, H), lambda s: (0, 0)),
            pl.BlockSpec((H,), lambda s: (0,)),
            pl.BlockSpec((H,), lambda s: (0,)),
        ],
        out_specs=pl.BlockSpec((B, BSS, H), lambda s: (0, s, 0)),
        out_shape=jax.ShapeDtypeStruct((B, S, H), jnp.float32),
    )(gathered, tt, proj_W, proj_b, pos_table,
      type_table, ln_gamma, ln_beta)

    return out


# speedup vs baseline: 1.0703x; 1.0703x over previous
import functools

import jax
import jax.numpy as jnp
from jax import lax
from jax.experimental import pallas as pl
from jax.experimental.pallas import tpu as pltpu
from jax.experimental.pallas import tpu_sc as plsc


def _sc_gather(table, idx):
    NB, NS_ = idx.shape
    NT = NB * NS_
    D = table.shape[1]
    info = plsc.get_sparse_core_info()
    NC, NS = info.num_cores, info.num_subcores
    NW = NC * NS
    per_w = NT // NW
    CH = 64
    n_ch = per_w // CH

    mesh = plsc.VectorSubcoreMesh(core_axis_name="c", subcore_axis_name="s")

    @functools.partial(
        pl.kernel,
        mesh=mesh,
        out_type=jax.ShapeDtypeStruct((NT, D), jnp.float32),
        scratch_types=(
            [pltpu.VMEM((per_w,), jnp.int32)]
            + [pltpu.VMEM((CH, D), jnp.float32)] * n_ch
            + [pltpu.SemaphoreType.DMA] * (2 * n_ch)
        ),
    )
    def gk(idx_hbm, table_hbm, out_hbm, idx_v, *scr):
        rows = scr[:n_ch]
        sg = scr[n_ch:2 * n_ch]
        sw = scr[2 * n_ch:]
        wid = lax.axis_index("s") * NC + lax.axis_index("c")
        base = wid * per_w
        row = base // NS_
        col = base - row * NS_
        pltpu.sync_copy(idx_hbm.at[row, pl.ds(col, per_w)], idx_v)
        gathers = [
            pltpu.async_copy(table_hbm.at[idx_v.at[pl.ds(j * CH, CH)]],
                             rows[j], sg[j])
            for j in range(n_ch)
        ]
        writes = []
        for j in range(n_ch):
            gathers[j].wait()
            writes.append(pltpu.async_copy(
                rows[j], out_hbm.at[pl.ds(base + j * CH, CH)], sw[j]))
        for w in writes:
            w.wait()

    return gk(idx, table)


def _tc_body(w_ref, tt_ref, pw_ref, pb_ref, pos_ref, tb_ref, g_ref, bt_ref,
             o_ref):
    B, BSS, E = w_ref.shape
    H = pos_ref.shape[-1]
    x = jnp.dot(w_ref[...].reshape(B * BSS, E), pw_ref[...],
                preferred_element_type=jnp.float32).reshape(B, BSS, H)
    x = x + pb_ref[...] + pos_ref[...][None]
    tid = tt_ref[0].astype(jnp.float32)
    x = x + tb_ref[0:1, :] + tid * (tb_ref[1:2, :] - tb_ref[0:1, :])
    mean = jnp.mean(x, axis=2, keepdims=True)
    xc = x - mean
    var = jnp.mean(xc * xc, axis=2, keepdims=True)
    inv = lax.rsqrt(var + 1e-5)
    o_ref[...] = (xc * inv) * g_ref[...] + bt_ref[...]


def kernel(input_ids, token_type_ids, word_table, proj_W, proj_b,
           pos_table, type_table, ln_gamma, ln_beta):
    B, S = input_ids.shape
    V, E = word_table.shape
    H = proj_W.shape[1]
    BSS = 512

    gathered = _sc_gather(word_table,
                          input_ids.astype(jnp.int32)).reshape(B, S, E)
    tt = (token_type_ids.astype(jnp.int8)
          .reshape(B, S // BSS, BSS).transpose(1, 0, 2)
          .reshape(S // BSS, B, BSS, 1))

    out = pl.pallas_call(
        _tc_body,
        grid=(S // BSS,),
        in_specs=[
            pl.BlockSpec((B, BSS, E), lambda s: (0, s, 0)),
            pl.BlockSpec((1, B, BSS, 1), lambda s: (s, 0, 0, 0)),
            pl.BlockSpec((E, H), lambda s: (0, 0)),
            pl.BlockSpec((H,), lambda s: (0,)),
            pl.BlockSpec((BSS, H), lambda s: (s, 0)),
            pl.BlockSpec((2, H), lambda s: (0, 0)),
            pl.BlockSpec((H,), lambda s: (0,)),
            pl.BlockSpec((H,), lambda s: (0,)),
        ],
        out_specs=pl.BlockSpec((B, BSS, H), lambda s: (0, s, 0)),
        out_shape=jax.ShapeDtypeStruct((B, S, H), jnp.float32),
    )(gathered, tt, proj_W, proj_b, pos_table,
      type_table, ln_gamma, ln_beta)

    return out
